# baseline (speedup 1.0000x reference)
import functools

import jax
import jax.numpy as jnp
from jax import lax
from jax.experimental import pallas as pl
from jax.experimental.pallas import tpu as pltpu
from jax.experimental.pallas import tpu_sc as plsc

_B, _D = 16384, 64
_NC, _NS, _L = 2, 16, 16
_NW = _NC * _NS
_RPW = _B // _NW
_CH = 64
_NCH = _RPW // _CH

_mesh = plsc.VectorSubcoreMesh(core_axis_name="c", subcore_axis_name="s")


@functools.partial(
    pl.kernel,
    out_type=jax.ShapeDtypeStruct((_B,), jnp.float32),
    mesh=_mesh,
    compiler_params=pltpu.CompilerParams(needs_layout_passes=False),
    scratch_types=[
        pltpu.VMEM((2, _CH, _D), jnp.float32),
        pltpu.VMEM((2, _CH, _D), jnp.float32),
        pltpu.VMEM((2, _CH, _D), jnp.float32),
        pltpu.VMEM((2, _CH, _D), jnp.float32),
        pltpu.VMEM((_RPW,), jnp.float32),
        pltpu.SemaphoreType.DMA,
        pltpu.SemaphoreType.DMA,
    ],
)
def _sc_kernel(gu_h, gi_h, gut_h, git_h, out_h,
               agu, agi, agut, agit, outv, sem0, sem1):
    wid = lax.axis_index("s") * _NC + lax.axis_index("c")
    base = wid * _RPW
    sems = (sem0, sem1)
    hv = ((gu_h, agu), (gi_h, agi), (gut_h, agut), (git_h, agit))

    def start(ci, slot):
        for h, v in hv:
            pltpu.async_copy(h.at[pl.ds(base + ci * _CH, _CH)],
                             v.at[slot], sems[slot])

    def wait(ci, slot):
        for h, v in hv:
            pltpu.make_async_copy(h.at[pl.ds(base + ci * _CH, _CH)],
                                  v.at[slot], sems[slot]).wait()

    def compute(ci, slot):
        a = agu.at[slot]
        b = agi.at[slot]
        c = agut.at[slot]
        d = agit.at[slot]
        obase = ci * _CH

        iota = lax.iota(jnp.int32, _L)

        def group(g, _):
            row0 = g * _L
            acc = jnp.zeros((_L,), jnp.float32)
            for r in range(_L):
                row = row0 + r
                q = None
                for j in range(_D // _L):
                    sl = pl.ds(j * _L, _L)
                    t = a[row, sl] * b[row, sl] + c[row, sl] * d[row, sl]
                    q = t if q is None else q + t
                s = jnp.sum(q)
                acc = jnp.where(iota == r, jnp.full((_L,), s), acc)
            outv[pl.ds(obase + row0, _L)] = acc
            return 0

        lax.fori_loop(0, _CH // _L, group, 0)

    start(0, 0)

    def pair(p, _):
        for k in (0, 1):
            ci = p * 2 + k
            wait(ci, k)

            @pl.when(ci + 1 < _NCH)
            def _():
                start(ci + 1, 1 - k)

            compute(ci, k)
        return 0

    lax.fori_loop(0, _NCH // 2, pair, 0)

    pltpu.sync_copy(outv, out_h.at[pl.ds(base, _RPW)])


def kernel(gu, gi, gut, git):
    return _sc_kernel(gu, gi, gut, git)
